# baseline (speedup 1.0000x reference)
import functools

import jax
import jax.numpy as jnp
from jax import lax
from jax.experimental import pallas as pl
from jax.experimental.pallas import tpu as pltpu
from jax.experimental.pallas import tpu_sc as plsc

_K = 32
_NBUF = 2


@functools.cache
def _build(B, V, Dp):
    info = plsc.get_sparse_core_info()
    nc, ns = info.num_cores, info.num_subcores
    nw = nc * ns
    assert B % (8 * nw) == 0 and _K % 8 == 0
    b_per_w = B // nw
    assert b_per_w % _K == 0
    n_chunks = b_per_w // _K

    mesh = plsc.VectorSubcoreMesh(core_axis_name="c", subcore_axis_name="s")

    def body(idx_hbm, table_hbm, out_hbm, idx_v, *bufs):
        rows = list(bufs[:_NBUF])
        gs = list(bufs[_NBUF:2 * _NBUF])
        ss = list(bufs[2 * _NBUF:])
        wid = lax.axis_index("s") * nc + lax.axis_index("c")
        base = wid * b_per_w
        pltpu.sync_copy(idx_hbm.at[pl.ds(base, b_per_w)], idx_v)

        def start_gather(i, b):
            pltpu.async_copy(
                table_hbm.at[idx_v.at[pl.ds(i * _K, _K)]], rows[b], gs[b]
            )

        def wait_gather(b):
            pltpu.make_async_copy(
                out_hbm.at[pl.ds(0, _K)], rows[b], gs[b]
            ).wait()

        def start_scatter(i, b):
            pltpu.async_copy(rows[b], out_hbm.at[pl.ds(base + i * _K, _K)], ss[b])

        def wait_scatter(b):
            pltpu.make_async_copy(
                rows[b], out_hbm.at[pl.ds(0, _K)], ss[b]
            ).wait()

        for b in range(_NBUF):
            start_gather(b, b)

        @pl.loop(0, n_chunks, step=_NBUF)
        def _(g):
            for b in range(_NBUF):
                i = g + b

                @pl.when(i < n_chunks)
                def _():
                    wait_gather(b)
                    start_scatter(i, b)

                    @pl.when(i + _NBUF < n_chunks)
                    def _():
                        wait_scatter(b)
                        start_gather(i + _NBUF, b)

        for b in range(_NBUF):
            wait_scatter(b)

    return pl.kernel(
        body,
        out_type=jax.ShapeDtypeStruct((B, Dp), jnp.float32),
        mesh=mesh,
        scratch_types=(
            [pltpu.VMEM((b_per_w,), jnp.int32)]
            + [pltpu.VMEM((_K, Dp), jnp.float32) for _ in range(_NBUF)]
            + [pltpu.SemaphoreType.DMA for _ in range(2 * _NBUF)]
        ),
    )


def kernel(idx, table):
    Bb, L = idx.shape
    V, D = table.shape
    pad = (-D) % 128
    Dp = D + pad
    table_p = jnp.pad(table, ((0, 0), (0, pad)))
    idx_flat = idx.reshape(-1).astype(jnp.int32)
    out = _build(Bb * L, V, Dp)(idx_flat, table_p)
    return out[:, :D].reshape(Bb, L, D)
